# baseline (speedup 1.0000x reference)
import jax
import jax.numpy as jnp
from jax.experimental import pallas as pl
from jax.experimental.pallas import tpu as pltpu

E, N, D, DFF = 8, 2048, 768, 3072
BT = 1024
FC = 1536
SB = 2


def _mlp_kernel(x_ref, w1_ref, b1_ref, w2_ref, b2_ref, o_ref):
    nf = DFF // FC
    sbt = BT // SB
    for s in range(SB):
        xs = x_ref[0, s * sbt:(s + 1) * sbt].astype(jnp.bfloat16)
        acc = jnp.broadcast_to(b2_ref[0], (sbt, D))
        for f in range(nf):
            lo, hi = f * FC, (f + 1) * FC
            a = jnp.dot(xs, w1_ref[0, :, lo:hi],
                        preferred_element_type=jnp.float32)
            g = jax.nn.gelu((a + b1_ref[0, :, lo:hi]).astype(jnp.bfloat16))
            acc = acc + jnp.dot(g, w2_ref[0, lo:hi, :],
                                preferred_element_type=jnp.float32)
        o_ref[0, s * sbt:(s + 1) * sbt] = acc


def kernel(x, W1, b1, W2, b2):
    B = x.shape[0]
    xe = x.reshape(E, N, D)
    b1r = b1.reshape(E, 1, DFF)
    b2r = b2.reshape(E, 1, D)

    out = pl.pallas_call(
        _mlp_kernel,
        grid=(E, N // BT),
        in_specs=[
            pl.BlockSpec((1, BT, D), lambda e, t: (e, t, 0)),
            pl.BlockSpec((1, D, DFF), lambda e, t: (e, 0, 0)),
            pl.BlockSpec((1, 1, DFF), lambda e, t: (e, 0, 0)),
            pl.BlockSpec((1, DFF, D), lambda e, t: (e, 0, 0)),
            pl.BlockSpec((1, 1, D), lambda e, t: (e, 0, 0)),
        ],
        out_specs=pl.BlockSpec((1, BT, D), lambda e, t: (e, t, 0)),
        out_shape=jax.ShapeDtypeStruct((E, N, D), jnp.float32),
        compiler_params=pltpu.CompilerParams(
            dimension_semantics=("parallel", "parallel"),
        ),
    )(xe, W1, b1r, W2, b2r)

    return out.reshape(B, E, N, D)


# speedup vs baseline: 1.0026x; 1.0026x over previous
import jax
import jax.numpy as jnp
from jax.experimental import pallas as pl
from jax.experimental.pallas import tpu as pltpu

E, N, D, DFF = 8, 2048, 768, 3072
BT = 1024
FC = 1536
SB = 2


def _mlp_kernel(x_ref, w1_ref, b1_ref, w2_ref, b2_ref, o_ref):
    sbt = BT // SB
    for s in range(SB):
        xs = x_ref[0, s * sbt:(s + 1) * sbt]
        acc = jnp.broadcast_to(b2_ref[0], (sbt, D))
        for f in range(DFF // FC):
            lo, hi = f * FC, (f + 1) * FC
            a = jnp.dot(xs, w1_ref[0, :, lo:hi],
                        preferred_element_type=jnp.float32)
            g = jax.nn.gelu((a + b1_ref[0, :, lo:hi]).astype(jnp.bfloat16))
            acc = acc + jnp.dot(g, w2_ref[0, lo:hi, :],
                                preferred_element_type=jnp.float32)
        o_ref[0, s * sbt:(s + 1) * sbt] = acc


def kernel(x, W1, b1, W2, b2):
    B = x.shape[0]
    xe = x.reshape(E, N, D)
    b1r = b1.reshape(E, 1, DFF)
    b2r = b2.reshape(E, 1, D)

    out = pl.pallas_call(
        _mlp_kernel,
        grid=(E, N // BT),
        in_specs=[
            pl.BlockSpec((1, BT, D), lambda e, t: (e, t, 0)),
            pl.BlockSpec((1, D, DFF), lambda e, t: (e, 0, 0)),
            pl.BlockSpec((1, 1, DFF), lambda e, t: (e, 0, 0)),
            pl.BlockSpec((1, DFF, D), lambda e, t: (e, 0, 0)),
            pl.BlockSpec((1, 1, D), lambda e, t: (e, 0, 0)),
        ],
        out_specs=pl.BlockSpec((1, BT, D), lambda e, t: (e, t, 0)),
        out_shape=jax.ShapeDtypeStruct((E, N, D), jnp.float32),
        compiler_params=pltpu.CompilerParams(
            dimension_semantics=("parallel", "parallel"),
        ),
    )(xe, W1, b1r, W2, b2r)

    return out.reshape(B, E, N, D)
